# Initial kernel scaffold; baseline (speedup 1.0000x reference)
import jax
import jax.numpy as jnp
from jax.experimental import pallas as pl


def kernel(x, edge_index, W1, b1, gn1_w, gn1_b, gn1_ms, W2, b2, gn2_w, gn2_b, gn2_ms, lW1, lb1, lW2, lb2):
    raise NotImplementedError("write your pallas kernel here")


# speedup vs baseline: 17.8669x; 17.8669x over previous
import functools

import jax
import jax.numpy as jnp
from jax import lax
from jax.experimental import pallas as pl
from jax.experimental.pallas import tpu as pltpu
from jax.experimental.pallas import tpu_sc as plsc

N = 10000
E = 320000
D = 128

NC = 2
NS = 16
NW = NC * NS
EPW = E // NW
K = 80
C = EPW // K
NP = 10240
RPT = NP // NS
RB = 128
DEGW = 16

_sc_mesh = plsc.VectorSubcoreMesh(core_axis_name="c", subcore_axis_name="s")


@functools.partial(
    pl.kernel,
    out_type=jax.ShapeDtypeStruct((NC, NP, DEGW), jnp.float32),
    mesh=_sc_mesh,
    compiler_params=pltpu.CompilerParams(use_tc_tiling_on_sc=False),
    scratch_types=[
        pltpu.VMEM_SHARED((NP, DEGW), jnp.float32),
        pltpu.VMEM((C, K), jnp.int32),
        pltpu.VMEM((K, DEGW), jnp.float32),
        pltpu.VMEM((RB, DEGW), jnp.float32),
    ],
)
def _sc_degree(dst_hbm, ones_hbm, zeros_hbm, out_hbm, acc, idx_v, ones_v, stage):
    c = lax.axis_index("c")
    s = lax.axis_index("s")
    wid = s * NC + c
    pltpu.sync_copy(zeros_hbm, stage)
    for i in range(RPT // RB):
        pltpu.sync_copy(stage, acc.at[pl.ds(s * RPT + i * RB, RB)])
    pltpu.sync_copy(ones_hbm, ones_v)
    pltpu.sync_copy(dst_hbm.at[wid], idx_v)
    plsc.subcore_barrier()

    def body(j, _):
        pltpu.sync_copy(ones_v, acc.at[idx_v.at[j]], add=True)
        return 0

    lax.fori_loop(0, C, body, 0)
    plsc.subcore_barrier()
    for i in range(RPT // RB):
        sl = pl.ds(s * RPT + i * RB, RB)
        pltpu.sync_copy(acc.at[sl], stage)
        pltpu.sync_copy(stage, out_hbm.at[c, sl])


@functools.partial(
    pl.kernel,
    out_type=jax.ShapeDtypeStruct((NC, NP, D), jnp.float32),
    mesh=_sc_mesh,
    compiler_params=pltpu.CompilerParams(use_tc_tiling_on_sc=False),
    scratch_types=[
        pltpu.VMEM_SHARED((NP, D), jnp.float32),
        pltpu.VMEM((C, K), jnp.int32),
        pltpu.VMEM((C, K), jnp.int32),
        pltpu.VMEM((K, D), jnp.float32),
        pltpu.VMEM((RB, D), jnp.float32),
    ],
)
def _sc_scatter(g_hbm, src_hbm, dst_hbm, zeros_hbm, out_hbm,
                acc, src_v, dst_v, rows, stage):
    c = lax.axis_index("c")
    s = lax.axis_index("s")
    wid = s * NC + c
    pltpu.sync_copy(zeros_hbm, stage)
    for i in range(RPT // RB):
        pltpu.sync_copy(stage, acc.at[pl.ds(s * RPT + i * RB, RB)])
    pltpu.sync_copy(src_hbm.at[wid], src_v)
    pltpu.sync_copy(dst_hbm.at[wid], dst_v)
    plsc.subcore_barrier()

    def body(j, _):
        pltpu.sync_copy(g_hbm.at[src_v.at[j]], rows)
        pltpu.sync_copy(rows, acc.at[dst_v.at[j]], add=True)
        return 0

    lax.fori_loop(0, C, body, 0)
    plsc.subcore_barrier()
    for i in range(RPT // RB):
        sl = pl.ds(s * RPT + i * RB, RB)
        pltpu.sync_copy(acc.at[sl], stage)
        pltpu.sync_copy(stage, out_hbm.at[c, sl])


R = 1000
GRID = N // R


def _dis(degA, degB):
    return lax.rsqrt(degA[:, :1] + degB[:, :1] + 1.0)


def _tc_pre_body(x_ref, w_ref, degA_ref, degB_ref, g_ref):
    g = jnp.dot(x_ref[...], w_ref[...], preferred_element_type=jnp.float32)
    g_ref[...] = g * _dis(degA_ref[...], degB_ref[...])


def _tc_post_body(accA_ref, accB_ref, g_ref, degA_ref, degB_ref, b_ref,
                  out_ref, sums_ref):
    i = pl.program_id(0)
    dis = _dis(degA_ref[...], degB_ref[...])
    out = (accA_ref[...] + accB_ref[...] + g_ref[...]) * dis + b_ref[...]
    out_ref[...] = out

    @pl.when(i == 0)
    def _():
        sums_ref[...] = jnp.zeros_like(sums_ref)

    sums_ref[0:1, :] += jnp.sum(out, axis=0, keepdims=True)
    sums_ref[1:2, :] += jnp.sum(out * out, axis=0, keepdims=True)


def _graph_norm(x, sums, w, b, ms, eps=1e-5):
    mean = sums[0:1, :] * (1.0 / N)
    ex2 = sums[1:2, :] * (1.0 / N)
    var = ex2 - mean * mean * ms * (2.0 - ms)
    return w * (x - mean * ms) / jnp.sqrt(var + eps) + b


def _tc_gn_mm_body(x_ref, sums_ref, w2_ref, degA_ref, degB_ref,
                   gnw_ref, gnb_ref, gnms_ref, g2_ref):
    y = jnp.maximum(
        _graph_norm(x_ref[...], sums_ref[...], gnw_ref[...], gnb_ref[...],
                    gnms_ref[...]), 0.0)
    h = jnp.dot(y, w2_ref[...], preferred_element_type=jnp.float32)
    g2_ref[...] = h * _dis(degA_ref[...], degB_ref[...])


def _tc_final_body(x_ref, sums_ref, gnw_ref, gnb_ref, gnms_ref,
                   lw1_ref, lb1_ref, lw2_ref, lb2_ref, y_ref):
    y = jnp.maximum(
        _graph_norm(x_ref[...], sums_ref[...], gnw_ref[...], gnb_ref[...],
                    gnms_ref[...]), 0.0)
    r = jnp.maximum(
        jnp.dot(y, lw1_ref[...], preferred_element_type=jnp.float32)
        + lb1_ref[...], 0.0)
    y_ref[...] = (jnp.dot(r, lw2_ref[...], preferred_element_type=jnp.float32)
                  + lb2_ref[...])


def _rows(shape):
    return pl.BlockSpec(shape, lambda i: (i, 0))


def _full(shape):
    return pl.BlockSpec(shape, lambda i: (0, 0))


_tc_pre = pl.pallas_call(
    _tc_pre_body,
    grid=(GRID,),
    in_specs=[_rows((R, D)), _full((D, D)), _rows((R, DEGW)), _rows((R, DEGW))],
    out_specs=_rows((R, D)),
    out_shape=jax.ShapeDtypeStruct((N, D), jnp.float32),
)

_tc_post = pl.pallas_call(
    _tc_post_body,
    grid=(GRID,),
    in_specs=[_rows((R, D)), _rows((R, D)), _rows((R, D)),
              _rows((R, DEGW)), _rows((R, DEGW)), _full((1, D))],
    out_specs=(_rows((R, D)), _full((8, D))),
    out_shape=(jax.ShapeDtypeStruct((N, D), jnp.float32),
               jax.ShapeDtypeStruct((8, D), jnp.float32)),
)

_tc_gn_mm = pl.pallas_call(
    _tc_gn_mm_body,
    grid=(GRID,),
    in_specs=[_rows((R, D)), _full((8, D)), _full((D, D)),
              _rows((R, DEGW)), _rows((R, DEGW)),
              _full((1, D)), _full((1, D)), _full((1, D))],
    out_specs=_rows((R, D)),
    out_shape=jax.ShapeDtypeStruct((N, D), jnp.float32),
)

_tc_final = pl.pallas_call(
    _tc_final_body,
    grid=(GRID,),
    in_specs=[_rows((R, D)), _full((8, D)),
              _full((1, D)), _full((1, D)), _full((1, D)),
              _full((D, D)), _full((1, D)), _full((D, D)), _full((1, D))],
    out_specs=_rows((R, D)),
    out_shape=jax.ShapeDtypeStruct((N, D), jnp.float32),
)


def kernel(x, edge_index, W1, b1, gn1_w, gn1_b, gn1_ms, W2, b2, gn2_w, gn2_b,
           gn2_ms, lW1, lb1, lW2, lb2):
    src = edge_index[0].reshape(NW, C, K)
    dst = edge_index[1].reshape(NW, C, K)
    ones16 = jnp.ones((K, DEGW), jnp.float32)
    zeros16 = jnp.zeros((RB, DEGW), jnp.float32)
    zerosD = jnp.zeros((RB, D), jnp.float32)
    row = lambda v: v.reshape(1, D)

    deg = _sc_degree(dst, ones16, zeros16)
    degA, degB = deg[0], deg[1]

    g1 = _tc_pre(x, W1, degA, degB)
    acc1 = _sc_scatter(g1, src, dst, zerosD)
    out1, sums1 = _tc_post(acc1[0], acc1[1], g1, degA, degB, row(b1))

    g2 = _tc_gn_mm(out1, sums1, W2, degA, degB,
                   row(gn1_w), row(gn1_b), row(gn1_ms))
    acc2 = _sc_scatter(g2, src, dst, zerosD)
    out2, sums2 = _tc_post(acc2[0], acc2[1], g2, degA, degB, row(b2))

    return _tc_final(out2, sums2, row(gn2_w), row(gn2_b), row(gn2_ms),
                     lW1, row(lb1), lW2, row(lb2))


# speedup vs baseline: 26.0845x; 1.4599x over previous
import functools

import jax
import jax.numpy as jnp
from jax import lax
from jax.experimental import pallas as pl
from jax.experimental.pallas import tpu as pltpu
from jax.experimental.pallas import tpu_sc as plsc

N = 10000
E = 320000
D = 128

NC = 2
NS = 16
NW = NC * NS
EPW = E // NW
K = 80
C = EPW // K
NP = 10240
RPT = NP // NS
RB = 128
DEGW = 16

_sc_mesh = plsc.VectorSubcoreMesh(core_axis_name="c", subcore_axis_name="s")


@functools.partial(
    pl.kernel,
    out_type=jax.ShapeDtypeStruct((NC, NP, DEGW), jnp.float32),
    mesh=_sc_mesh,
    compiler_params=pltpu.CompilerParams(use_tc_tiling_on_sc=False),
    scratch_types=[
        pltpu.VMEM_SHARED((NP, DEGW), jnp.float32),
        pltpu.VMEM((C, K), jnp.int32),
        pltpu.VMEM((K, DEGW), jnp.float32),
        pltpu.VMEM((RB, DEGW), jnp.float32),
    ],
)
def _sc_degree(dst_hbm, ones_hbm, zeros_hbm, out_hbm, acc, idx_v, ones_v, stage):
    c = lax.axis_index("c")
    s = lax.axis_index("s")
    wid = s * NC + c
    pltpu.sync_copy(zeros_hbm, stage)
    for i in range(RPT // RB):
        pltpu.sync_copy(stage, acc.at[pl.ds(s * RPT + i * RB, RB)])
    pltpu.sync_copy(ones_hbm, ones_v)
    pltpu.sync_copy(dst_hbm.at[wid], idx_v)
    plsc.subcore_barrier()

    def body(j, _):
        pltpu.sync_copy(ones_v, acc.at[idx_v.at[j]], add=True)
        return 0

    lax.fori_loop(0, C, body, 0)
    plsc.subcore_barrier()
    for i in range(RPT // RB):
        sl = pl.ds(s * RPT + i * RB, RB)
        pltpu.sync_copy(acc.at[sl], stage)
        pltpu.sync_copy(stage, out_hbm.at[c, sl])


@functools.partial(
    pl.kernel,
    out_type=jax.ShapeDtypeStruct((NC, NP, D), jnp.float32),
    mesh=_sc_mesh,
    compiler_params=pltpu.CompilerParams(use_tc_tiling_on_sc=False),
    scratch_types=[
        pltpu.VMEM_SHARED((NP, D), jnp.float32),
        pltpu.VMEM((C, K), jnp.int32),
        pltpu.VMEM((C, K), jnp.int32),
        pltpu.VMEM((K, D), jnp.float32),
        pltpu.VMEM((K, D), jnp.float32),
        pltpu.SemaphoreType.DMA,
        pltpu.SemaphoreType.DMA,
    ],
)
def _sc_scatter(g_hbm, src_hbm, dst_hbm, zeros_hbm, out_hbm,
                acc, src_v, dst_v, rows0, rows1, sem0, sem1):
    c = lax.axis_index("c")
    s = lax.axis_index("s")
    wid = s * NC + c
    pltpu.sync_copy(zeros_hbm, rows0)
    for i in range(RPT // K):
        pltpu.sync_copy(rows0, acc.at[pl.ds(s * RPT + i * K, K)])
    pltpu.sync_copy(src_hbm.at[wid], src_v)
    pltpu.sync_copy(dst_hbm.at[wid], dst_v)
    plsc.subcore_barrier()

    pltpu.async_copy(g_hbm.at[src_v.at[0]], rows0, sem0)

    def body(jj, _):
        j0 = 2 * jj
        d1 = pltpu.async_copy(g_hbm.at[src_v.at[j0 + 1]], rows1, sem1)
        pltpu.make_async_copy(g_hbm.at[src_v.at[j0]], rows0, sem0).wait()
        pltpu.sync_copy(rows0, acc.at[dst_v.at[j0]], add=True)
        pltpu.async_copy(g_hbm.at[src_v.at[j0 + 2]], rows0, sem0)
        d1.wait()
        pltpu.sync_copy(rows1, acc.at[dst_v.at[j0 + 1]], add=True)
        return 0

    lax.fori_loop(0, (C - 1) // 2, body, 0)
    pltpu.make_async_copy(g_hbm.at[src_v.at[C - 1]], rows0, sem0).wait()
    pltpu.sync_copy(rows0, acc.at[dst_v.at[C - 1]], add=True)
    plsc.subcore_barrier()
    for i in range(RPT // K):
        sl = pl.ds(s * RPT + i * K, K)
        pltpu.sync_copy(acc.at[sl], rows0)
        pltpu.sync_copy(rows0, out_hbm.at[c, sl])


R = 1000
GRID = N // R


def _dis(degA, degB):
    return lax.rsqrt(degA[:, :1] + degB[:, :1] + 1.0)


def _tc_pre_body(x_ref, w_ref, degA_ref, degB_ref, g_ref):
    g = jnp.dot(x_ref[...], w_ref[...], preferred_element_type=jnp.float32)
    g_ref[...] = g * _dis(degA_ref[...], degB_ref[...])


def _tc_post_body(accA_ref, accB_ref, g_ref, degA_ref, degB_ref, b_ref,
                  out_ref, sums_ref):
    i = pl.program_id(0)
    dis = _dis(degA_ref[...], degB_ref[...])
    out = (accA_ref[...] + accB_ref[...] + g_ref[...]) * dis + b_ref[...]
    out_ref[...] = out

    @pl.when(i == 0)
    def _():
        sums_ref[...] = jnp.zeros_like(sums_ref)

    sums_ref[0:1, :] += jnp.sum(out, axis=0, keepdims=True)
    sums_ref[1:2, :] += jnp.sum(out * out, axis=0, keepdims=True)


def _graph_norm(x, sums, w, b, ms, eps=1e-5):
    mean = sums[0:1, :] * (1.0 / N)
    ex2 = sums[1:2, :] * (1.0 / N)
    var = ex2 - mean * mean * ms * (2.0 - ms)
    return w * (x - mean * ms) / jnp.sqrt(var + eps) + b


def _tc_gn_mm_body(x_ref, sums_ref, w2_ref, degA_ref, degB_ref,
                   gnw_ref, gnb_ref, gnms_ref, g2_ref):
    y = jnp.maximum(
        _graph_norm(x_ref[...], sums_ref[...], gnw_ref[...], gnb_ref[...],
                    gnms_ref[...]), 0.0)
    h = jnp.dot(y, w2_ref[...], preferred_element_type=jnp.float32)
    g2_ref[...] = h * _dis(degA_ref[...], degB_ref[...])


def _tc_final_body(x_ref, sums_ref, gnw_ref, gnb_ref, gnms_ref,
                   lw1_ref, lb1_ref, lw2_ref, lb2_ref, y_ref):
    y = jnp.maximum(
        _graph_norm(x_ref[...], sums_ref[...], gnw_ref[...], gnb_ref[...],
                    gnms_ref[...]), 0.0)
    r = jnp.maximum(
        jnp.dot(y, lw1_ref[...], preferred_element_type=jnp.float32)
        + lb1_ref[...], 0.0)
    y_ref[...] = (jnp.dot(r, lw2_ref[...], preferred_element_type=jnp.float32)
                  + lb2_ref[...])


def _rows(shape):
    return pl.BlockSpec(shape, lambda i: (i, 0))


def _full(shape):
    return pl.BlockSpec(shape, lambda i: (0, 0))


_tc_pre = pl.pallas_call(
    _tc_pre_body,
    grid=(GRID,),
    in_specs=[_rows((R, D)), _full((D, D)), _rows((R, DEGW)), _rows((R, DEGW))],
    out_specs=_rows((R, D)),
    out_shape=jax.ShapeDtypeStruct((N, D), jnp.float32),
)

_tc_post = pl.pallas_call(
    _tc_post_body,
    grid=(GRID,),
    in_specs=[_rows((R, D)), _rows((R, D)), _rows((R, D)),
              _rows((R, DEGW)), _rows((R, DEGW)), _full((1, D))],
    out_specs=(_rows((R, D)), _full((8, D))),
    out_shape=(jax.ShapeDtypeStruct((N, D), jnp.float32),
               jax.ShapeDtypeStruct((8, D), jnp.float32)),
)

_tc_gn_mm = pl.pallas_call(
    _tc_gn_mm_body,
    grid=(GRID,),
    in_specs=[_rows((R, D)), _full((8, D)), _full((D, D)),
              _rows((R, DEGW)), _rows((R, DEGW)),
              _full((1, D)), _full((1, D)), _full((1, D))],
    out_specs=_rows((R, D)),
    out_shape=jax.ShapeDtypeStruct((N, D), jnp.float32),
)

_tc_final = pl.pallas_call(
    _tc_final_body,
    grid=(GRID,),
    in_specs=[_rows((R, D)), _full((8, D)),
              _full((1, D)), _full((1, D)), _full((1, D)),
              _full((D, D)), _full((1, D)), _full((D, D)), _full((1, D))],
    out_specs=_rows((R, D)),
    out_shape=jax.ShapeDtypeStruct((N, D), jnp.float32),
)


def kernel(x, edge_index, W1, b1, gn1_w, gn1_b, gn1_ms, W2, b2, gn2_w, gn2_b,
           gn2_ms, lW1, lb1, lW2, lb2):
    src = edge_index[0].reshape(NW, C, K)
    dst = edge_index[1].reshape(NW, C, K)
    ones16 = jnp.ones((K, DEGW), jnp.float32)
    zeros16 = jnp.zeros((RB, DEGW), jnp.float32)
    zerosD = jnp.zeros((K, D), jnp.float32)
    row = lambda v: v.reshape(1, D)

    deg = _sc_degree(dst, ones16, zeros16)
    degA, degB = deg[0], deg[1]

    g1 = _tc_pre(x, W1, degA, degB)
    acc1 = _sc_scatter(g1, src, dst, zerosD)
    out1, sums1 = _tc_post(acc1[0], acc1[1], g1, degA, degB, row(b1))

    g2 = _tc_gn_mm(out1, sums1, W2, degA, degB,
                   row(gn1_w), row(gn1_b), row(gn1_ms))
    acc2 = _sc_scatter(g2, src, dst, zerosD)
    out2, sums2 = _tc_post(acc2[0], acc2[1], g2, degA, degB, row(b2))

    return _tc_final(out2, sums2, row(gn2_w), row(gn2_b), row(gn2_ms),
                     lW1, row(lb1), lW2, row(lb2))


# speedup vs baseline: 26.1227x; 1.0015x over previous
import functools

import jax
import jax.numpy as jnp
from jax import lax
from jax.experimental import pallas as pl
from jax.experimental.pallas import tpu as pltpu
from jax.experimental.pallas import tpu_sc as plsc

N = 10000
E = 320000
D = 128

NC = 2
NS = 16
NW = NC * NS
EPW = E // NW
K = 40
NBUF = 5
C = EPW // K
NP = 10240
RPT = NP // NS
RB = 128
DEGW = 16

_sc_mesh = plsc.VectorSubcoreMesh(core_axis_name="c", subcore_axis_name="s")


@functools.partial(
    pl.kernel,
    out_type=jax.ShapeDtypeStruct((NC, NP, DEGW), jnp.float32),
    mesh=_sc_mesh,
    compiler_params=pltpu.CompilerParams(use_tc_tiling_on_sc=False),
    scratch_types=[
        pltpu.VMEM_SHARED((NP, DEGW), jnp.float32),
        pltpu.VMEM((C, K), jnp.int32),
        pltpu.VMEM((K, DEGW), jnp.float32),
        pltpu.VMEM((RB, DEGW), jnp.float32),
    ],
)
def _sc_degree(dst_hbm, ones_hbm, zeros_hbm, out_hbm, acc, idx_v, ones_v, stage):
    c = lax.axis_index("c")
    s = lax.axis_index("s")
    wid = s * NC + c
    pltpu.sync_copy(zeros_hbm, stage)
    for i in range(RPT // RB):
        pltpu.sync_copy(stage, acc.at[pl.ds(s * RPT + i * RB, RB)])
    pltpu.sync_copy(ones_hbm, ones_v)
    pltpu.sync_copy(dst_hbm.at[wid], idx_v)
    plsc.subcore_barrier()

    def body(j, _):
        pltpu.sync_copy(ones_v, acc.at[idx_v.at[j]], add=True)
        return 0

    lax.fori_loop(0, C, body, 0)
    plsc.subcore_barrier()
    for i in range(RPT // RB):
        sl = pl.ds(s * RPT + i * RB, RB)
        pltpu.sync_copy(acc.at[sl], stage)
        pltpu.sync_copy(stage, out_hbm.at[c, sl])


@functools.partial(
    pl.kernel,
    out_type=jax.ShapeDtypeStruct((NC, NP, D), jnp.float32),
    mesh=_sc_mesh,
    compiler_params=pltpu.CompilerParams(use_tc_tiling_on_sc=False),
    scratch_types=[
        pltpu.VMEM_SHARED((NP, D), jnp.float32),
        pltpu.VMEM((C, K), jnp.int32),
        pltpu.VMEM((C, K), jnp.int32),
    ]
    + [pltpu.VMEM((K, D), jnp.float32) for _ in range(NBUF)]
    + [pltpu.SemaphoreType.DMA for _ in range(2 * NBUF)],
)
def _sc_scatter(g_hbm, src_hbm, dst_hbm, zeros_hbm, out_hbm,
                acc, src_v, dst_v, *bufs_and_sems):
    rows = bufs_and_sems[:NBUF]
    gsem = bufs_and_sems[NBUF:2 * NBUF]
    ssem = bufs_and_sems[2 * NBUF:]
    c = lax.axis_index("c")
    s = lax.axis_index("s")
    wid = s * NC + c
    pltpu.sync_copy(src_hbm.at[wid], src_v)
    pltpu.sync_copy(dst_hbm.at[wid], dst_v)
    pltpu.sync_copy(zeros_hbm, rows[0])
    for i in range(RPT // K):
        pltpu.sync_copy(rows[0], acc.at[pl.ds(s * RPT + i * K, K)])
    for t in range(NBUF):
        pltpu.async_copy(g_hbm.at[src_v.at[t]], rows[t], gsem[t])
    plsc.subcore_barrier()

    def body(i, _):
        ds = []
        for t in range(NBUF):
            j = NBUF * i + t
            pltpu.make_async_copy(g_hbm.at[src_v.at[j]], rows[t], gsem[t]).wait()
            ds.append(pltpu.async_copy(rows[t], acc.at[dst_v.at[j]], ssem[t],
                                       add=True))
        for t in range(NBUF):
            ds[t].wait()

            @pl.when(i < C // NBUF - 1)
            def _():
                j2 = NBUF * i + NBUF + t
                pltpu.async_copy(g_hbm.at[src_v.at[j2]], rows[t], gsem[t])
        return 0

    lax.fori_loop(0, C // NBUF, body, 0)
    plsc.subcore_barrier()
    for i in range(RPT // K):
        sl = pl.ds(s * RPT + i * K, K)
        pltpu.sync_copy(acc.at[sl], rows[0])
        pltpu.sync_copy(rows[0], out_hbm.at[c, sl])


R = 1000
GRID = N // R


def _dis(degA, degB):
    return lax.rsqrt(degA[:, :1] + degB[:, :1] + 1.0)


def _tc_pre_body(x_ref, w_ref, degA_ref, degB_ref, g_ref):
    g = jnp.dot(x_ref[...], w_ref[...], preferred_element_type=jnp.float32)
    g_ref[...] = g * _dis(degA_ref[...], degB_ref[...])


def _tc_post_body(accA_ref, accB_ref, g_ref, degA_ref, degB_ref, b_ref,
                  out_ref, sums_ref):
    i = pl.program_id(0)
    dis = _dis(degA_ref[...], degB_ref[...])
    out = (accA_ref[...] + accB_ref[...] + g_ref[...]) * dis + b_ref[...]
    out_ref[...] = out

    @pl.when(i == 0)
    def _():
        sums_ref[...] = jnp.zeros_like(sums_ref)

    sums_ref[0:1, :] += jnp.sum(out, axis=0, keepdims=True)
    sums_ref[1:2, :] += jnp.sum(out * out, axis=0, keepdims=True)


def _graph_norm(x, sums, w, b, ms, eps=1e-5):
    mean = sums[0:1, :] * (1.0 / N)
    ex2 = sums[1:2, :] * (1.0 / N)
    var = ex2 - mean * mean * ms * (2.0 - ms)
    return w * (x - mean * ms) / jnp.sqrt(var + eps) + b


def _tc_gn_mm_body(x_ref, sums_ref, w2_ref, degA_ref, degB_ref,
                   gnw_ref, gnb_ref, gnms_ref, g2_ref):
    y = jnp.maximum(
        _graph_norm(x_ref[...], sums_ref[...], gnw_ref[...], gnb_ref[...],
                    gnms_ref[...]), 0.0)
    h = jnp.dot(y, w2_ref[...], preferred_element_type=jnp.float32)
    g2_ref[...] = h * _dis(degA_ref[...], degB_ref[...])


def _tc_final_body(x_ref, sums_ref, gnw_ref, gnb_ref, gnms_ref,
                   lw1_ref, lb1_ref, lw2_ref, lb2_ref, y_ref):
    y = jnp.maximum(
        _graph_norm(x_ref[...], sums_ref[...], gnw_ref[...], gnb_ref[...],
                    gnms_ref[...]), 0.0)
    r = jnp.maximum(
        jnp.dot(y, lw1_ref[...], preferred_element_type=jnp.float32)
        + lb1_ref[...], 0.0)
    y_ref[...] = (jnp.dot(r, lw2_ref[...], preferred_element_type=jnp.float32)
                  + lb2_ref[...])


def _rows(shape):
    return pl.BlockSpec(shape, lambda i: (i, 0))


def _full(shape):
    return pl.BlockSpec(shape, lambda i: (0, 0))


_tc_pre = pl.pallas_call(
    _tc_pre_body,
    grid=(GRID,),
    in_specs=[_rows((R, D)), _full((D, D)), _rows((R, DEGW)), _rows((R, DEGW))],
    out_specs=_rows((R, D)),
    out_shape=jax.ShapeDtypeStruct((N, D), jnp.float32),
)

_tc_post = pl.pallas_call(
    _tc_post_body,
    grid=(GRID,),
    in_specs=[_rows((R, D)), _rows((R, D)), _rows((R, D)),
              _rows((R, DEGW)), _rows((R, DEGW)), _full((1, D))],
    out_specs=(_rows((R, D)), _full((8, D))),
    out_shape=(jax.ShapeDtypeStruct((N, D), jnp.float32),
               jax.ShapeDtypeStruct((8, D), jnp.float32)),
)

_tc_gn_mm = pl.pallas_call(
    _tc_gn_mm_body,
    grid=(GRID,),
    in_specs=[_rows((R, D)), _full((8, D)), _full((D, D)),
              _rows((R, DEGW)), _rows((R, DEGW)),
              _full((1, D)), _full((1, D)), _full((1, D))],
    out_specs=_rows((R, D)),
    out_shape=jax.ShapeDtypeStruct((N, D), jnp.float32),
)

_tc_final = pl.pallas_call(
    _tc_final_body,
    grid=(GRID,),
    in_specs=[_rows((R, D)), _full((8, D)),
              _full((1, D)), _full((1, D)), _full((1, D)),
              _full((D, D)), _full((1, D)), _full((D, D)), _full((1, D))],
    out_specs=_rows((R, D)),
    out_shape=jax.ShapeDtypeStruct((N, D), jnp.float32),
)


def kernel(x, edge_index, W1, b1, gn1_w, gn1_b, gn1_ms, W2, b2, gn2_w, gn2_b,
           gn2_ms, lW1, lb1, lW2, lb2):
    src = edge_index[0].reshape(NW, C, K)
    dst = edge_index[1].reshape(NW, C, K)
    ones16 = jnp.ones((K, DEGW), jnp.float32)
    zeros16 = jnp.zeros((RB, DEGW), jnp.float32)
    zerosD = jnp.zeros((K, D), jnp.float32)
    row = lambda v: v.reshape(1, D)

    deg = _sc_degree(dst, ones16, zeros16)
    degA, degB = deg[0], deg[1]

    g1 = _tc_pre(x, W1, degA, degB)
    acc1 = _sc_scatter(g1, src, dst, zerosD)
    out1, sums1 = _tc_post(acc1[0], acc1[1], g1, degA, degB, row(b1))

    g2 = _tc_gn_mm(out1, sums1, W2, degA, degB,
                   row(gn1_w), row(gn1_b), row(gn1_ms))
    acc2 = _sc_scatter(g2, src, dst, zerosD)
    out2, sums2 = _tc_post(acc2[0], acc2[1], g2, degA, degB, row(b2))

    return _tc_final(out2, sums2, row(gn2_w), row(gn2_b), row(gn2_ms),
                     lW1, row(lb1), lW2, row(lb2))


# speedup vs baseline: 28.5357x; 1.0924x over previous
import functools

import jax
import jax.numpy as jnp
from jax import lax
from jax.experimental import pallas as pl
from jax.experimental.pallas import tpu as pltpu
from jax.experimental.pallas import tpu_sc as plsc

N = 10000
E = 320000
D = 128

NC = 2
NS = 16
NW = NC * NS
EPW = E // NW
K = 40
NBUF = 5
C = EPW // K
NP = 10240
RPT = NP // NS
RB = 128
DEGW = 16

_sc_mesh = plsc.VectorSubcoreMesh(core_axis_name="c", subcore_axis_name="s")


@functools.partial(
    pl.kernel,
    out_type=(jax.ShapeDtypeStruct((NP, DEGW), jnp.float32),
              jax.ShapeDtypeStruct((NP, DEGW), jnp.float32)),
    mesh=_sc_mesh,
    compiler_params=pltpu.CompilerParams(use_tc_tiling_on_sc=False),
    scratch_types=[
        pltpu.VMEM_SHARED((NP, DEGW), jnp.float32),
        pltpu.VMEM((C, K), jnp.int32),
        pltpu.VMEM((K, DEGW), jnp.float32),
        pltpu.VMEM((RB, DEGW), jnp.float32),
    ],
)
def _sc_degree(edges_hbm, ones_hbm, zeros_hbm, outA, outB, acc, idx_v, ones_v,
               stage):
    c = lax.axis_index("c")
    s = lax.axis_index("s")
    wid = s * NC + c
    pltpu.sync_copy(zeros_hbm, stage)
    for i in range(RPT // RB):
        pltpu.sync_copy(stage, acc.at[pl.ds(s * RPT + i * RB, RB)])
    pltpu.sync_copy(ones_hbm, ones_v)
    pltpu.sync_copy(edges_hbm.at[1, wid], idx_v)
    plsc.subcore_barrier()

    def body(j, _):
        pltpu.sync_copy(ones_v, acc.at[idx_v.at[j]], add=True)
        return 0

    lax.fori_loop(0, C, body, 0)
    plsc.subcore_barrier()
    for i in range(RPT // RB):
        sl = pl.ds(s * RPT + i * RB, RB)
        pltpu.sync_copy(acc.at[sl], stage)

        @pl.when(c == 0)
        def _():
            pltpu.sync_copy(stage, outA.at[sl])

        @pl.when(c == 1)
        def _():
            pltpu.sync_copy(stage, outB.at[sl])


@functools.partial(
    pl.kernel,
    out_type=(jax.ShapeDtypeStruct((NP, D), jnp.float32),
              jax.ShapeDtypeStruct((NP, D), jnp.float32)),
    mesh=_sc_mesh,
    compiler_params=pltpu.CompilerParams(use_tc_tiling_on_sc=False),
    scratch_types=[
        pltpu.VMEM_SHARED((NP, D), jnp.float32),
        pltpu.VMEM((C, K), jnp.int32),
        pltpu.VMEM((C, K), jnp.int32),
    ]
    + [pltpu.VMEM((K, D), jnp.float32) for _ in range(NBUF)]
    + [pltpu.SemaphoreType.DMA for _ in range(2 * NBUF)],
)
def _sc_scatter(g_hbm, edges_hbm, zeros_hbm, outA, outB,
                acc, src_v, dst_v, *bufs_and_sems):
    rows = bufs_and_sems[:NBUF]
    gsem = bufs_and_sems[NBUF:2 * NBUF]
    ssem = bufs_and_sems[2 * NBUF:]
    c = lax.axis_index("c")
    s = lax.axis_index("s")
    wid = s * NC + c
    pltpu.sync_copy(edges_hbm.at[0, wid], src_v)
    pltpu.sync_copy(edges_hbm.at[1, wid], dst_v)
    pltpu.sync_copy(zeros_hbm, rows[0])
    for i in range(RPT // K):
        pltpu.sync_copy(rows[0], acc.at[pl.ds(s * RPT + i * K, K)])
    for t in range(NBUF):
        pltpu.async_copy(g_hbm.at[src_v.at[t]], rows[t], gsem[t])
    plsc.subcore_barrier()

    def body(i, _):
        ds = []
        for t in range(NBUF):
            j = NBUF * i + t
            pltpu.make_async_copy(g_hbm.at[src_v.at[j]], rows[t], gsem[t]).wait()
            ds.append(pltpu.async_copy(rows[t], acc.at[dst_v.at[j]], ssem[t],
                                       add=True))
        for t in range(NBUF):
            ds[t].wait()

            @pl.when(i < C // NBUF - 1)
            def _():
                j2 = NBUF * i + NBUF + t
                pltpu.async_copy(g_hbm.at[src_v.at[j2]], rows[t], gsem[t])
        return 0

    lax.fori_loop(0, C // NBUF, body, 0)
    plsc.subcore_barrier()
    for i in range(RPT // K):
        sl = pl.ds(s * RPT + i * K, K)
        pltpu.sync_copy(acc.at[sl], rows[0])

        @pl.when(c == 0)
        def _():
            pltpu.sync_copy(rows[0], outA.at[sl])

        @pl.when(c == 1)
        def _():
            pltpu.sync_copy(rows[0], outB.at[sl])


R = 1000
GRID = N // R


def _dis(degA, degB):
    return lax.rsqrt(degA[:, :1] + degB[:, :1] + 1.0)


def _tc_pre_body(x_ref, w_ref, degA_ref, degB_ref, g_ref):
    g = jnp.dot(x_ref[...], w_ref[...], preferred_element_type=jnp.float32)
    g_ref[...] = g * _dis(degA_ref[...], degB_ref[...])


def _tc_post_body(accA_ref, accB_ref, g_ref, degA_ref, degB_ref, b_ref,
                  out_ref, sums_ref):
    i = pl.program_id(0)
    dis = _dis(degA_ref[...], degB_ref[...])
    out = (accA_ref[...] + accB_ref[...] + g_ref[...]) * dis + b_ref[...]
    out_ref[...] = out

    @pl.when(i == 0)
    def _():
        sums_ref[...] = jnp.zeros_like(sums_ref)

    sums_ref[0:1, :] += jnp.sum(out, axis=0, keepdims=True)
    sums_ref[1:2, :] += jnp.sum(out * out, axis=0, keepdims=True)


def _graph_norm(x, sums, w, b, ms, eps=1e-5):
    mean = sums[0:1, :] * (1.0 / N)
    ex2 = sums[1:2, :] * (1.0 / N)
    var = ex2 - mean * mean * ms * (2.0 - ms)
    return w * (x - mean * ms) / jnp.sqrt(var + eps) + b


def _tc_gn_mm_body(x_ref, sums_ref, w2_ref, degA_ref, degB_ref,
                   gnw_ref, gnb_ref, gnms_ref, g2_ref):
    y = jnp.maximum(
        _graph_norm(x_ref[...], sums_ref[...], gnw_ref[...], gnb_ref[...],
                    gnms_ref[...]), 0.0)
    h = jnp.dot(y, w2_ref[...], preferred_element_type=jnp.float32)
    g2_ref[...] = h * _dis(degA_ref[...], degB_ref[...])


def _tc_final_body(x_ref, sums_ref, gnw_ref, gnb_ref, gnms_ref,
                   lw1_ref, lb1_ref, lw2_ref, lb2_ref, y_ref):
    y = jnp.maximum(
        _graph_norm(x_ref[...], sums_ref[...], gnw_ref[...], gnb_ref[...],
                    gnms_ref[...]), 0.0)
    r = jnp.maximum(
        jnp.dot(y, lw1_ref[...], preferred_element_type=jnp.float32)
        + lb1_ref[...], 0.0)
    y_ref[...] = (jnp.dot(r, lw2_ref[...], preferred_element_type=jnp.float32)
                  + lb2_ref[...])


def _rows(shape):
    return pl.BlockSpec(shape, lambda i: (i, 0))


def _full(shape):
    return pl.BlockSpec(shape, lambda i: (0, 0))


_tc_pre = pl.pallas_call(
    _tc_pre_body,
    grid=(GRID,),
    in_specs=[_rows((R, D)), _full((D, D)), _rows((R, DEGW)), _rows((R, DEGW))],
    out_specs=_rows((R, D)),
    out_shape=jax.ShapeDtypeStruct((N, D), jnp.float32),
)

_tc_post = pl.pallas_call(
    _tc_post_body,
    grid=(GRID,),
    in_specs=[_rows((R, D)), _rows((R, D)), _rows((R, D)),
              _rows((R, DEGW)), _rows((R, DEGW)), _full((1, D))],
    out_specs=(_rows((R, D)), _full((8, D))),
    out_shape=(jax.ShapeDtypeStruct((N, D), jnp.float32),
               jax.ShapeDtypeStruct((8, D), jnp.float32)),
)

_tc_gn_mm = pl.pallas_call(
    _tc_gn_mm_body,
    grid=(GRID,),
    in_specs=[_rows((R, D)), _full((8, D)), _full((D, D)),
              _rows((R, DEGW)), _rows((R, DEGW)),
              _full((1, D)), _full((1, D)), _full((1, D))],
    out_specs=_rows((R, D)),
    out_shape=jax.ShapeDtypeStruct((N, D), jnp.float32),
)

_tc_final = pl.pallas_call(
    _tc_final_body,
    grid=(GRID,),
    in_specs=[_rows((R, D)), _full((8, D)),
              _full((1, D)), _full((1, D)), _full((1, D)),
              _full((D, D)), _full((1, D)), _full((D, D)), _full((1, D))],
    out_specs=_rows((R, D)),
    out_shape=jax.ShapeDtypeStruct((N, D), jnp.float32),
)


def kernel(x, edge_index, W1, b1, gn1_w, gn1_b, gn1_ms, W2, b2, gn2_w, gn2_b,
           gn2_ms, lW1, lb1, lW2, lb2):
    edges = edge_index.reshape(2, NW, C, K)
    ones16 = jnp.ones((K, DEGW), jnp.float32)
    zeros16 = jnp.zeros((RB, DEGW), jnp.float32)
    zerosD = jnp.zeros((K, D), jnp.float32)
    row = lambda v: v.reshape(1, D)

    degA, degB = _sc_degree(edges, ones16, zeros16)

    g1 = _tc_pre(x, W1, degA, degB)
    a1, b1acc = _sc_scatter(g1, edges, zerosD)
    out1, sums1 = _tc_post(a1, b1acc, g1, degA, degB, row(b1))

    g2 = _tc_gn_mm(out1, sums1, W2, degA, degB,
                   row(gn1_w), row(gn1_b), row(gn1_ms))
    a2, b2acc = _sc_scatter(g2, edges, zerosD)
    out2, sums2 = _tc_post(a2, b2acc, g2, degA, degB, row(b2))

    return _tc_final(out2, sums2, row(gn2_w), row(gn2_b), row(gn2_ms),
                     lW1, row(lb1), lW2, row(lb2))
